# baseline (speedup 1.0000x reference)
import jax
import jax.numpy as jnp
from jax import lax
from jax.experimental import pallas as pl
from jax.experimental.pallas import tpu as pltpu
from jax.experimental.pallas import tpu_sc as plsc

NN = 10000
EE = 320000
DD = 128
NC = 2
NS = 16
NW = NC * NS
NPAD = 10240
ROWS_PER_TILE = NPAD // NS
DUMP = NN
EPAD = NW * 10240
B = 128
K1_BATCHES = EPAD // NW // B
K3_BATCHES = EPAD // NS // B
DEGW = 128
RB = 1024
GRID1 = NPAD // RB
GRID2 = 2 * NPAD // RB

_MESH = plsc.VectorSubcoreMesh(core_axis_name="c", subcore_axis_name="s",
                               num_cores=NC, num_subcores=NS)


def _deg_body(dst_hbm, e0_hbm, zdeg_hbm, deg_out, deg_sp, idx_v, val_v):
    c = lax.axis_index("c")
    s = lax.axis_index("s")
    w = c * NS + s
    pltpu.sync_copy(zdeg_hbm, deg_sp.at[pl.ds(s * ROWS_PER_TILE, ROWS_PER_TILE)])
    pltpu.sync_copy(e0_hbm, val_v)
    pltpu.sync_copy(dst_hbm.at[w], idx_v)
    plsc.subcore_barrier()

    def body(j, carry):
        pltpu.sync_copy(val_v, deg_sp.at[idx_v.at[j]], add=True)
        return carry

    lax.fori_loop(0, K1_BATCHES, body, 0)
    plsc.subcore_barrier()
    pltpu.sync_copy(deg_sp.at[pl.ds(s * ROWS_PER_TILE, ROWS_PER_TILE)],
                    deg_out.at[w])


def _deg_call(dst_k1, e0, zdeg):
    f = pl.kernel(
        _deg_body,
        out_type=jax.ShapeDtypeStruct((NW, ROWS_PER_TILE, DEGW), jnp.float32),
        mesh=_MESH,
        scratch_types=[
            pltpu.VMEM_SHARED((NPAD, DEGW), jnp.float32),
            pltpu.VMEM((K1_BATCHES, B), jnp.int32),
            pltpu.VMEM((B, DEGW), jnp.float32),
        ],
    )
    return f(dst_k1, e0, zdeg)


BA = 64
NB = 2
LOOK = 1
CHUNK = 40
EDGES_PER_TILE = EPAD // NS
NCH = EDGES_PER_TILE // (CHUNK * BA)


def _agg_body(src_hbm, dst_hbm, ys_hbm, zrows_hbm, acc_out,
              acc_sp, src_c, dst_c, rows4, sem_g, sem_s):
    c = lax.axis_index("c")
    s = lax.axis_index("s")
    w = c * NS + s
    pltpu.sync_copy(zrows_hbm, acc_sp.at[pl.ds(s * ROWS_PER_TILE, ROWS_PER_TILE)])
    plsc.subcore_barrier()

    def chunk_body(ko, carry):
        pltpu.sync_copy(src_hbm.at[w * NCH + ko], src_c)
        pltpu.sync_copy(dst_hbm.at[s * NCH + ko], dst_c)
        for p in range(LOOK):
            pltpu.async_copy(ys_hbm.at[src_c.at[p]], rows4.at[p], sem_g)

        def group(q, carry2):
            for t in range(NB):
                jj = NB * q + t
                b_cur = t
                b_nxt = (t + LOOK) % NB

                @pl.when(jj + LOOK < CHUNK)
                def _():
                    pltpu.async_copy(ys_hbm.at[src_c.at[jj + LOOK]],
                                     rows4.at[b_nxt], sem_g)

                pltpu.make_async_copy(ys_hbm.at[src_c.at[jj]],
                                      rows4.at[b_cur], sem_g).wait()
            return carry2

        lax.fori_loop(0, CHUNK // NB, group, 0)
        return carry

    lax.fori_loop(0, NCH, chunk_body, 0)
    plsc.subcore_barrier()
    pltpu.sync_copy(acc_sp.at[pl.ds(s * ROWS_PER_TILE, ROWS_PER_TILE)],
                    acc_out.at[w])


def _agg_call(src_k3, dst_k3, ys_all, zrows):
    f = pl.kernel(
        _agg_body,
        out_type=jax.ShapeDtypeStruct((NW, ROWS_PER_TILE, DD), jnp.float32),
        mesh=_MESH,
        scratch_types=[
            pltpu.VMEM_SHARED((NPAD, DD), jnp.float32),
            pltpu.VMEM((CHUNK, BA), jnp.int32),
            pltpu.VMEM((CHUNK, BA), jnp.int32),
            pltpu.VMEM((NB, BA, 256), jnp.float32),
            pltpu.SemaphoreType.DMA,
            pltpu.SemaphoreType.DMA,
        ],
        name="agg_probe",
    )
    return f(src_k3, dst_k3, ys_all, zrows)


def _mm_body(x_ref, w_ref, degT_ref, ys_ref, dinv_ref):
    deg = degT_ref[:, 0:1] + degT_ref[:, 1:2] + 1.0
    dinv = lax.rsqrt(jnp.maximum(deg, 1e-12))
    y = jnp.dot(x_ref[...], w_ref[...], preferred_element_type=jnp.float32)
    ys_ref[...] = y * dinv
    dinv_ref[...] = dinv


def _mm_call(x_all, W_gcn, degT):
    return pl.pallas_call(
        _mm_body,
        grid=(GRID2,),
        in_specs=[
            pl.BlockSpec((RB, DD), lambda i: (i, 0)),
            pl.BlockSpec((DD, DD), lambda i: (0, 0)),
            pl.BlockSpec((RB, 2), lambda i: (lax.rem(i, GRID1), 0)),
        ],
        out_specs=[
            pl.BlockSpec((RB, DD), lambda i: (i, 0)),
            pl.BlockSpec((RB, 1), lambda i: (lax.rem(i, GRID1), 0)),
        ],
        out_shape=[
            jax.ShapeDtypeStruct((2 * NPAD, DD), jnp.float32),
            jax.ShapeDtypeStruct((NPAD, 1), jnp.float32),
        ],
    )(x_all, W_gcn, degT)


def _act_body(acc_ref, ys_ref, dinv_ref, b_ref, a_ref, h_ref, colsum_ref):
    i = pl.program_id(0)
    g = (acc_ref[...] + ys_ref[...]) * dinv_ref[...] + b_ref[...]
    a = a_ref[0, 0]
    h = jnp.where(g > 0, g, a * g)
    h_ref[...] = h

    @pl.when(i < GRID1)
    def _():
        rows = i * RB + lax.broadcasted_iota(jnp.int32, (RB, 1), 0)
        hm = jnp.where(rows < NN, h, 0.0)
        part = jnp.sum(hm, axis=0, keepdims=True)

        @pl.when(i == 0)
        def _():
            colsum_ref[...] = part

        @pl.when(i > 0)
        def _():
            colsum_ref[...] = colsum_ref[...] + part


def _act_call(acc_all, ys_all, dinv, b_gcn, prelu_a):
    return pl.pallas_call(
        _act_body,
        grid=(GRID2,),
        in_specs=[
            pl.BlockSpec((RB, DD), lambda i: (i, 0)),
            pl.BlockSpec((RB, DD), lambda i: (i, 0)),
            pl.BlockSpec((RB, 1), lambda i: (lax.rem(i, GRID1), 0)),
            pl.BlockSpec((1, DD), lambda i: (0, 0)),
            pl.BlockSpec((1, 1), lambda i: (0, 0)),
        ],
        out_specs=[
            pl.BlockSpec((RB, DD), lambda i: (i, 0)),
            pl.BlockSpec((1, DD), lambda i: (0, 0)),
        ],
        out_shape=[
            jax.ShapeDtypeStruct((2 * NPAD, DD), jnp.float32),
            jax.ShapeDtypeStruct((1, DD), jnp.float32),
        ],
    )(acc_all, ys_all, dinv, b_gcn, prelu_a)


def _score_body(h_ref, colsum_ref, wbilT_ref, bb_ref, s_ref):
    cs = colsum_ref[...] * (1.0 / NN)
    cvec = 1.0 / (1.0 + jnp.exp(-cs))
    v = jnp.dot(cvec, wbilT_ref[...], preferred_element_type=jnp.float32)
    s_ref[...] = jnp.sum(h_ref[...] * v, axis=1, keepdims=True) + bb_ref[...]


def _score_call(h_all, colsum, wbilT, b_bil):
    return pl.pallas_call(
        _score_body,
        grid=(GRID2,),
        in_specs=[
            pl.BlockSpec((RB, DD), lambda i: (i, 0)),
            pl.BlockSpec((1, DD), lambda i: (0, 0)),
            pl.BlockSpec((DD, DD), lambda i: (0, 0)),
            pl.BlockSpec((1, 1), lambda i: (0, 0)),
        ],
        out_specs=pl.BlockSpec((RB, 1), lambda i: (i, 0)),
        out_shape=jax.ShapeDtypeStruct((2 * NPAD, 1), jnp.float32),
    )(h_all, colsum, wbilT, b_bil)


def kernel(seq1, seq2, adj, W_gcn, b_gcn, prelu_a, W_bil, b_bil):
    src = adj[0].astype(jnp.int32)
    dst = adj[1].astype(jnp.int32)
    pad_e = EPAD - EE
    srcp = jnp.concatenate([src, jnp.zeros((pad_e,), jnp.int32)])
    dstp = jnp.concatenate([dst, jnp.full((pad_e,), DUMP, jnp.int32)])

    dst_k1 = dstp.reshape(NW, K1_BATCHES, B)
    src_k3 = (srcp.reshape(1, NS, EDGES_PER_TILE)
              + (jnp.arange(NC, dtype=jnp.int32) * NPAD).reshape(NC, 1, 1)
              ).reshape(NW * NCH, CHUNK, BA)
    dst_k3 = dstp.reshape(NS * NCH, CHUNK, BA)

    e0 = jnp.zeros((B, DEGW), jnp.float32).at[:, 0].set(1.0)
    zdeg = jnp.zeros((ROWS_PER_TILE, DEGW), jnp.float32)
    zrows = jnp.zeros((ROWS_PER_TILE, DD), jnp.float32)

    deg_out = _deg_call(dst_k1, e0, zdeg)
    degT = deg_out.reshape(NC, NPAD, DEGW)[:, :, 0].T

    rpad = NPAD - NN
    x_all = jnp.concatenate([
        jnp.concatenate([seq1, jnp.zeros((rpad, DD), jnp.float32)]),
        jnp.concatenate([seq2, jnp.zeros((rpad, DD), jnp.float32)]),
    ])
    ys_all, dinv = _mm_call(x_all, W_gcn, degT)

    acc_out = _agg_call(src_k3, dst_k3, jnp.concatenate([ys_all, ys_all], axis=1), zrows)
    acc_all = acc_out.reshape(2 * NPAD, DD)

    h_all, colsum = _act_call(acc_all, ys_all, dinv,
                              b_gcn.reshape(1, DD),
                              prelu_a.reshape(1, 1))

    s_all = _score_call(h_all, colsum, W_bil[0].T, b_bil.reshape(1, 1))

    logits = jnp.concatenate([s_all[:NN, 0], s_all[NPAD:NPAD + NN, 0]])
    return logits[None, :]


# speedup vs baseline: 1.2009x; 1.2009x over previous
import jax
import jax.numpy as jnp
from jax import lax
from jax.experimental import pallas as pl
from jax.experimental.pallas import tpu as pltpu
from jax.experimental.pallas import tpu_sc as plsc

NN = 10000
EE = 320000
DD = 128
NC = 2
NS = 16
NW = NC * NS
NPAD = 10240
RPT = NPAD // NS
DUMP = NN
EPAD = NW * 10240
B = 128
K1_BATCHES = EPAD // NW // B
DEGW = 128

_MESH = plsc.VectorSubcoreMesh(core_axis_name="c", subcore_axis_name="s",
                               num_cores=NC, num_subcores=NS)


def _deg_body(dst_hbm, e0_hbm, zdeg_hbm, deg_out, deg_sp, idx_v, val_v):
    c = lax.axis_index("c")
    s = lax.axis_index("s")
    w = c * NS + s
    pltpu.sync_copy(zdeg_hbm, deg_sp.at[pl.ds(s * RPT, RPT)])
    pltpu.sync_copy(e0_hbm, val_v)
    pltpu.sync_copy(dst_hbm.at[w], idx_v)
    plsc.subcore_barrier()

    def body(j, carry):
        pltpu.sync_copy(val_v, deg_sp.at[idx_v.at[j]], add=True)
        return carry

    lax.fori_loop(0, K1_BATCHES, body, 0)
    plsc.subcore_barrier()
    pltpu.sync_copy(deg_sp.at[pl.ds(s * RPT, RPT)], deg_out.at[w])


def _deg_call(dst_k1, e0, zdeg):
    f = pl.kernel(
        _deg_body,
        out_type=jax.ShapeDtypeStruct((NW, RPT, DEGW), jnp.float32),
        mesh=_MESH,
        scratch_types=[
            pltpu.VMEM_SHARED((NPAD, DEGW), jnp.float32),
            pltpu.VMEM((K1_BATCHES, B), jnp.int32),
            pltpu.VMEM((B, DEGW), jnp.float32),
        ],
        name="deg",
    )
    return f(dst_k1, e0, zdeg)


BA = 80
NB = 4
LOOK = 2
CHUNK = 32
EDGES_PER_TILE = EPAD // NS
NCH = EDGES_PER_TILE // (CHUNK * BA)


def _agg_body(src_hbm, dst_hbm, ys_hbm, zrows_hbm, acc_out,
              acc_sp, src_c, dst_c, rows4, sem_g, sem_s):
    c = lax.axis_index("c")
    s = lax.axis_index("s")
    w = c * NS + s
    pltpu.sync_copy(zrows_hbm, acc_sp.at[pl.ds(s * RPT, RPT)])
    plsc.subcore_barrier()

    def chunk_body(ko, carry):
        pltpu.sync_copy(src_hbm.at[w * NCH + ko], src_c)
        pltpu.sync_copy(dst_hbm.at[s * NCH + ko], dst_c)
        for p in range(LOOK):
            pltpu.async_copy(ys_hbm.at[src_c.at[p]], rows4.at[p], sem_g)

        def group(q, carry2):
            for t in range(NB):
                jj = NB * q + t
                b_cur = t
                b_nxt = (t + LOOK) % NB

                @pl.when(jj >= NB - LOOK)
                def _():
                    pltpu.make_async_copy(rows4.at[b_nxt],
                                          acc_sp.at[dst_c.at[jj - (NB - LOOK)]],
                                          sem_s).wait()

                @pl.when(jj + LOOK < CHUNK)
                def _():
                    pltpu.async_copy(ys_hbm.at[src_c.at[jj + LOOK]],
                                     rows4.at[b_nxt], sem_g)

                pltpu.make_async_copy(ys_hbm.at[src_c.at[jj]],
                                      rows4.at[b_cur], sem_g).wait()
                pltpu.async_copy(rows4.at[b_cur], acc_sp.at[dst_c.at[jj]],
                                 sem_s, add=True)
            return carry2

        lax.fori_loop(0, CHUNK // NB, group, 0)
        for r in range(NB - LOOK, 0, -1):
            pltpu.make_async_copy(rows4.at[(CHUNK - r) % NB],
                                  acc_sp.at[dst_c.at[CHUNK - r]],
                                  sem_s).wait()
        return carry

    lax.fori_loop(0, NCH, chunk_body, 0)
    plsc.subcore_barrier()
    pltpu.sync_copy(acc_sp.at[pl.ds(s * RPT, RPT)], acc_out.at[w])


def _agg_call(src_k3, dst_k3, ys_all, zrows):
    f = pl.kernel(
        _agg_body,
        out_type=jax.ShapeDtypeStruct((NW, RPT, DD), jnp.float32),
        mesh=_MESH,
        scratch_types=[
            pltpu.VMEM_SHARED((NPAD, DD), jnp.float32),
            pltpu.VMEM((CHUNK, BA), jnp.int32),
            pltpu.VMEM((CHUNK, BA), jnp.int32),
            pltpu.VMEM((NB, BA, DD), jnp.float32),
            pltpu.SemaphoreType.DMA,
            pltpu.SemaphoreType.DMA,
        ],
        name="agg",
    )
    return f(src_k3, dst_k3, ys_all, zrows)


RBK = RPT
NBK = 2 * NPAD // RBK


def _mm_body(x1_ref, x2_ref, w_ref, y_ref):
    j = pl.program_id(0)
    x = jnp.where(j < NBK // 2, x1_ref[...], x2_ref[...])
    y_ref[...] = jnp.dot(x, w_ref[...], preferred_element_type=jnp.float32)


def _mm_call(seq1, seq2, W_gcn):
    nxb = NBK // 2 - 1
    return pl.pallas_call(
        _mm_body,
        grid=(NBK,),
        in_specs=[
            pl.BlockSpec((RBK, DD), lambda j: (lax.min(j, nxb), 0)),
            pl.BlockSpec((RBK, DD),
                         lambda j: (lax.min(lax.max(j - (nxb + 1), 0), nxb), 0)),
            pl.BlockSpec((DD, DD), lambda j: (0, 0)),
        ],
        out_specs=pl.BlockSpec((RBK, DD), lambda j: (j, 0)),
        out_shape=jax.ShapeDtypeStruct((2 * NPAD, DD), jnp.float32),
    )(seq1, seq2, W_gcn)


def _scale_body(y_ref, degT_ref, ys_ref):
    deg = degT_ref[:, 0:1] + degT_ref[:, 1:2] + 1.0
    dinv = lax.rsqrt(jnp.maximum(deg, 1e-12))
    ys_ref[...] = y_ref[...] * dinv


def _scale_call(y_raw, degT):
    return pl.pallas_call(
        _scale_body,
        grid=(NBK,),
        in_specs=[
            pl.BlockSpec((RBK, DD), lambda j: (j, 0)),
            pl.BlockSpec((RBK, 2), lambda j: (lax.rem(j, NBK // 2), 0)),
        ],
        out_specs=pl.BlockSpec((RBK, DD), lambda j: (j, 0)),
        out_shape=jax.ShapeDtypeStruct((2 * NPAD, DD), jnp.float32),
    )(y_raw, degT)


def _epi_body(acc_ref, ys_ref, degT_ref, b_ref, a_ref, wbt_ref, bb_ref,
              s_ref, hbuf, colsum):
    j = pl.program_id(0)

    @pl.when(j < NBK)
    def _():
        deg = degT_ref[:, 0:1] + degT_ref[:, 1:2] + 1.0
        dinv = lax.rsqrt(jnp.maximum(deg, 1e-12))
        g = (acc_ref[...] + ys_ref[...]) * dinv + b_ref[...]
        a = a_ref[0, 0]
        h = jnp.where(g > 0, g, a * g)
        hbuf[pl.ds(j * RBK, RBK)] = h

        @pl.when(j < NBK // 2)
        def _():
            rows = j * RBK + lax.broadcasted_iota(jnp.int32, (RBK, 1), 0)
            part = jnp.sum(jnp.where(rows < NN, h, 0.0), axis=0,
                           keepdims=True)

            @pl.when(j == 0)
            def _():
                colsum[...] = part

            @pl.when(j > 0)
            def _():
                colsum[...] = colsum[...] + part

    @pl.when(j >= NBK)
    def _():
        jj = j - NBK
        cs = colsum[...] * (1.0 / NN)
        cvec = 1.0 / (1.0 + jnp.exp(-cs))
        v = jnp.dot(cvec, wbt_ref[...], preferred_element_type=jnp.float32)
        h = hbuf[pl.ds(jj * RBK, RBK)]
        s_ref[...] = jnp.sum(h * v, axis=1, keepdims=True) + bb_ref[...]


def _epi_call(acc_all, ys_all, degT, b_gcn, prelu_a, wbilT, b_bil):
    nb1 = NBK - 1
    return pl.pallas_call(
        _epi_body,
        grid=(2 * NBK,),
        in_specs=[
            pl.BlockSpec((RBK, DD), lambda j: (lax.min(j, nb1), 0)),
            pl.BlockSpec((RBK, DD), lambda j: (lax.min(j, nb1), 0)),
            pl.BlockSpec((RBK, 2), lambda j: (lax.rem(j, NBK // 2), 0)),
            pl.BlockSpec((1, DD), lambda j: (0, 0)),
            pl.BlockSpec((1, 1), lambda j: (0, 0)),
            pl.BlockSpec((DD, DD), lambda j: (0, 0)),
            pl.BlockSpec((1, 1), lambda j: (0, 0)),
        ],
        out_specs=pl.BlockSpec((RBK, 1), lambda j: (lax.rem(j, NBK), 0)),
        out_shape=jax.ShapeDtypeStruct((2 * NPAD, 1), jnp.float32),
        scratch_shapes=[
            pltpu.VMEM((2 * NPAD, DD), jnp.float32),
            pltpu.VMEM((1, DD), jnp.float32),
        ],
    )(acc_all, ys_all, degT, b_gcn, prelu_a, wbilT, b_bil)


def kernel(seq1, seq2, adj, W_gcn, b_gcn, prelu_a, W_bil, b_bil):
    src = adj[0].astype(jnp.int32)
    dst = adj[1].astype(jnp.int32)
    pad_e = EPAD - EE
    srcp = jnp.concatenate([src, jnp.zeros((pad_e,), jnp.int32)])
    dstp = jnp.concatenate([dst, jnp.full((pad_e,), DUMP, jnp.int32)])

    dst_k1 = dstp.reshape(NW, K1_BATCHES, B)
    src_k3 = (srcp.reshape(1, NS, EDGES_PER_TILE)
              + (jnp.arange(NC, dtype=jnp.int32) * NPAD).reshape(NC, 1, 1)
              ).reshape(NW * NCH, CHUNK, BA)
    dst_k3 = dstp.reshape(NS * NCH, CHUNK, BA)

    e0 = jnp.zeros((B, DEGW), jnp.float32).at[:, 0].set(1.0)
    zdeg = jnp.zeros((RPT, DEGW), jnp.float32)
    zrows = jnp.zeros((RPT, DD), jnp.float32)

    deg_out = _deg_call(dst_k1, e0, zdeg)
    y_raw = _mm_call(seq1, seq2, W_gcn)

    degT = deg_out[:, :, 0].reshape(NC, NPAD).T
    ys_all = _scale_call(y_raw, degT)

    acc_out = _agg_call(src_k3, dst_k3, ys_all, zrows)
    acc_all = acc_out.reshape(2 * NPAD, DD)

    s_all = _epi_call(acc_all, ys_all, degT,
                      b_gcn.reshape(1, DD), prelu_a.reshape(1, 1),
                      W_bil[0].T, b_bil.reshape(1, 1))

    logits = jnp.concatenate([s_all[:NN, 0], s_all[NPAD:NPAD + NN, 0]])
    return logits[None, :]
